# baseline (speedup 1.0000x reference)
import functools

import jax
import jax.numpy as jnp
from jax import lax
from jax.experimental import pallas as pl
from jax.experimental.pallas import tpu as pltpu
from jax.experimental.pallas import tpu_sc as plsc

NC = 2
NS = 16
L = 16
NW = NC * NS
HALF = 100


@functools.cache
def _emb_call(B, T, D, V):
    seq_w = B // NW
    vpr = D // L

    mesh = plsc.VectorSubcoreMesh(core_axis_name="c", subcore_axis_name="s")

    @functools.partial(
        pl.kernel,
        out_type=jax.ShapeDtypeStruct((B, T, D), jnp.float32),
        mesh=mesh,
        compiler_params=pltpu.CompilerParams(use_tc_tiling_on_sc=False),
        scratch_types=[
            pltpu.VMEM((seq_w * 2, HALF), jnp.int32),
            pltpu.VMEM((T, D), jnp.float32),
            pltpu.VMEM((T, 2 * D), jnp.float32),
            pltpu.VMEM((T, 2 * D), jnp.float32),
            pltpu.VMEM((T, D), jnp.float32),
            pltpu.VMEM((T, D), jnp.float32),
            pltpu.SemaphoreType.DMA,
            pltpu.SemaphoreType.DMA,
            pltpu.SemaphoreType.DMA,
            pltpu.SemaphoreType.DMA,
        ],
    )
    def emb(idx_hbm, tok_hbm, pos_hbm, out_hbm, idx_v, pos_v,
            g0, g1, o0, o1, sg0, sg1, so0, so1):
        wid = lax.axis_index("s") * NC + lax.axis_index("c")
        pltpu.sync_copy(idx_hbm.at[wid], idx_v)
        pltpu.sync_copy(pos_hbm, pos_v)

        gbuf, obuf = (g0, g1), (o0, o1)
        gsem, osem = (sg0, sg1), (so0, so1)

        def fire_gather(j):
            b = j % 2
            return [
                pltpu.make_async_copy(
                    tok_hbm.at[idx_v.at[2 * j + h]],
                    gbuf[b].at[pl.ds(h * HALF, HALF)],
                    gsem[b],
                ) for h in range(2)
            ]
        for cp in [c for j in range(2) for c in fire_gather(j)]:
            cp.start()

        gh = {0: fire_gather(0), 1: fire_gather(1)}
        oh = {}
        for j in range(seq_w):
            b = j % 2
            for cp in gh[j]:
                cp.wait()
            if j >= 2:
                oh[j - 2].wait()

            def row_body(r, carry, _g=gbuf[b], _o=obuf[b]):
                for q in range(vpr):
                    s = pl.ds(q * L, L)
                    _o[r, s] = _g[r, s] + pos_v[r, s]
                return carry

            lax.fori_loop(0, T, row_body, 0)

            oh[j] = pltpu.make_async_copy(
                obuf[b], out_hbm.at[wid * seq_w + j], osem[b])
            oh[j].start()
            if j + 2 < seq_w:
                gh[j + 2] = fire_gather(j + 2)
                for cp in gh[j + 2]:
                    cp.start()
        oh[seq_w - 2].wait()
        oh[seq_w - 1].wait()

    return emb


def kernel(idx, tok_table, pos_table):
    B, T = idx.shape
    V, D = tok_table.shape
    assert B % NW == 0 and T == 2 * HALF and D % L == 0
    tok_pad = jnp.pad(tok_table, ((0, 0), (0, D)))
    idx3 = idx.astype(jnp.int32).reshape(NW, (B // NW) * 2, HALF)
    return _emb_call(B, T, D, V)(idx3, tok_pad, pos_table)


# speedup vs baseline: 1.1272x; 1.1272x over previous
import functools

import jax
import jax.numpy as jnp
from jax import lax
from jax.experimental import pallas as pl
from jax.experimental.pallas import tpu as pltpu
from jax.experimental.pallas import tpu_sc as plsc

NC = 2
NS = 16
L = 16
NW = NC * NS
HALF = 100


@functools.cache
def _emb_call(B, T, D, V):
    seq_w = B // NW
    vpr = D // L

    mesh = plsc.VectorSubcoreMesh(core_axis_name="c", subcore_axis_name="s")

    @functools.partial(
        pl.kernel,
        out_type=jax.ShapeDtypeStruct((B * T // 8, 8, 2 * D), jnp.float32),
        mesh=mesh,
        compiler_params=pltpu.CompilerParams(use_tc_tiling_on_sc=False),
        scratch_types=[
            pltpu.VMEM((seq_w * 2, HALF), jnp.int32),
            pltpu.VMEM((T, D), jnp.float32),
            pltpu.VMEM((T, D), jnp.float32),
            pltpu.VMEM((T, D), jnp.float32),
            pltpu.VMEM((T // 8, 8, D), jnp.float32),
            pltpu.VMEM((T // 8, 8, D), jnp.float32),
            pltpu.SemaphoreType.DMA,
            pltpu.SemaphoreType.DMA,
            pltpu.SemaphoreType.DMA,
            pltpu.SemaphoreType.DMA,
        ],
    )
    def emb(idx_hbm, tok_hbm, pos_hbm, out_hbm, idx_v, pos_v,
            g0, g1, o0, o1, sg0, sg1, so0, so1):
        wid = lax.axis_index("s") * NC + lax.axis_index("c")
        pltpu.sync_copy(idx_hbm.at[wid], idx_v)
        pltpu.sync_copy(pos_hbm, pos_v)

        gbuf, obuf = (g0, g1), (o0, o1)
        gsem, osem = (sg0, sg1), (so0, so1)

        def fire_gather(j):
            b = j % 2
            return [
                pltpu.make_async_copy(
                    tok_hbm.at[idx_v.at[2 * j + h]],
                    gbuf[b].at[pl.ds(h * HALF, HALF)],
                    gsem[b],
                ) for h in range(2)
            ]
        for cp in [c for j in range(2) for c in fire_gather(j)]:
            cp.start()

        gh = {0: fire_gather(0), 1: fire_gather(1)}
        oh = {}
        for j in range(seq_w):
            b = j % 2
            for cp in gh[j]:
                cp.wait()
            if j >= 2:
                oh[j - 2].wait()

            def row_body(r, carry, _g=gbuf[b], _o=obuf[b]):
                rt, rr = lax.shift_right_logical(r, 3), r & 7
                for q in range(vpr):
                    s = pl.ds(q * L, L)
                    _o[rt, rr, s] = _g[r, s] + pos_v[r, s]
                return carry

            lax.fori_loop(0, T, row_body, 0)

            oh[j] = pltpu.make_async_copy(
                obuf[b],
                out_hbm.at[pl.ds((wid * seq_w + j) * (T // 8), T // 8), :,
                           pl.ds(0, D)],
                osem[b])
            oh[j].start()
            if j + 2 < seq_w:
                gh[j + 2] = fire_gather(j + 2)
                for cp in gh[j + 2]:
                    cp.start()
        oh[seq_w - 2].wait()
        oh[seq_w - 1].wait()

    return emb


def kernel(idx, tok_table, pos_table):
    B, T = idx.shape
    V, D = tok_table.shape
    assert B % NW == 0 and T == 2 * HALF and D % L == 0
    idx3 = idx.astype(jnp.int32).reshape(NW, (B // NW) * 2, HALF)
    out6 = _emb_call(B, T, D, V)(idx3, tok_table, pos_table)
    return out6[:, :, :D].reshape(B, T, D)
